# Initial kernel scaffold; baseline (speedup 1.0000x reference)
import jax
import jax.numpy as jnp
from jax.experimental import pallas as pl


def kernel(x1, edge_index1, x2, edge_index2, W0, b0, W1, b1):
    raise NotImplementedError("write your pallas kernel here")


# speedup vs baseline: 8.8538x; 8.8538x over previous
import functools

import jax
import jax.numpy as jnp
from jax import lax
from jax.experimental import pallas as pl
from jax.experimental.pallas import tpu as pltpu
from jax.experimental.pallas import tpu_sc as plsc

N = 10000
NP = 10240
D = 128
E = 320000
NT = 16
CH = 128
NCHUNK = 157
EPT = CH * NCHUNK
E_PAD = EPT * NT
PAD = E_PAD - E
RPT = NP // NT
DUMMY = N

_MESH = plsc.VectorSubcoreMesh(core_axis_name="c", subcore_axis_name="s",
                               num_cores=2, num_subcores=NT)


def _agg_body(y_hbm, src_hbm, dst_hbm, out_hbm, sidx, didx, rows, acc, sem):
    c = lax.axis_index("c")
    s = lax.axis_index("s")

    pltpu.sync_copy(y_hbm.at[pl.ds(c * NP + s * RPT, RPT)],
                    acc.at[pl.ds(s * RPT, RPT)])
    plsc.subcore_barrier()

    e0 = c * E_PAD + s * EPT

    def body(i, _):
        base = pl.multiple_of(e0 + i * CH, CH)
        pltpu.sync_copy(src_hbm.at[pl.ds(base, CH)], sidx)
        pltpu.sync_copy(dst_hbm.at[pl.ds(base, CH)], didx)
        pltpu.async_copy(y_hbm.at[sidx], rows, sem).wait()
        pltpu.sync_copy(rows, acc.at[didx], add=True)
        return 0

    lax.fori_loop(0, NCHUNK, body, 0)
    plsc.subcore_barrier()
    pltpu.sync_copy(acc.at[pl.ds(s * RPT, RPT)],
                    out_hbm.at[pl.ds(c * NP + s * RPT, RPT)])


_agg_call = functools.partial(
    pl.kernel,
    out_type=jax.ShapeDtypeStruct((2 * NP, D), jnp.float32),
    mesh=_MESH,
    scratch_types=[
        pltpu.VMEM((CH,), jnp.int32),
        pltpu.VMEM((CH,), jnp.int32),
        pltpu.VMEM((CH, D), jnp.float32),
        pltpu.VMEM_SHARED((NP, D), jnp.float32),
        pltpu.SemaphoreType.DMA,
    ],
)(_agg_body)


_BR = 2048


def _dinv(deg_ref):
    return lax.rsqrt(deg_ref[:, 0:1])


def _mm_scale_body(x_ref, w_ref, deg_ref, y_ref):
    y_ref[...] = _dinv(deg_ref) * jnp.dot(
        x_ref[...], w_ref[...], preferred_element_type=jnp.float32)


def _mid_body(agg_ref, deg_ref, b_ref, w_ref, y_ref):
    dinv = _dinv(deg_ref)
    h = jnp.maximum(dinv * agg_ref[...] + b_ref[...], 0.0)
    y_ref[...] = dinv * jnp.dot(h, w_ref[...], preferred_element_type=jnp.float32)


def _final_body(agg_ref, deg_ref, b_ref, z_ref):
    z_ref[...] = _dinv(deg_ref) * agg_ref[...] + b_ref[...]


def _row_spec(w):
    return pl.BlockSpec((_BR, w), lambda i: (i, 0))


def _fixed_spec(h, w):
    return pl.BlockSpec((h, w), lambda i: (0, 0))


_mm_scale = pl.pallas_call(
    _mm_scale_body,
    grid=(2 * NP // _BR,),
    in_specs=[_row_spec(D), _fixed_spec(D, D), _row_spec(D)],
    out_specs=_row_spec(D),
    out_shape=jax.ShapeDtypeStruct((2 * NP, D), jnp.float32),
)

_mid = pl.pallas_call(
    _mid_body,
    grid=(2 * NP // _BR,),
    in_specs=[_row_spec(D), _row_spec(D), _fixed_spec(1, D), _fixed_spec(D, D)],
    out_specs=_row_spec(D),
    out_shape=jax.ShapeDtypeStruct((2 * NP, D), jnp.float32),
)

_final = pl.pallas_call(
    _final_body,
    grid=(2 * NP // _BR,),
    in_specs=[_row_spec(D), _row_spec(D), _fixed_spec(1, D)],
    out_specs=_row_spec(D),
    out_shape=jax.ShapeDtypeStruct((2 * NP, D), jnp.float32),
)


def kernel(x1, edge_index1, x2, edge_index2, W0, b0, W1, b1):
    pad_src = jnp.zeros((PAD,), jnp.int32)
    pad_dst = jnp.full((PAD,), DUMMY, jnp.int32)
    src = jnp.concatenate([edge_index1[0].astype(jnp.int32), pad_src,
                           edge_index2[0].astype(jnp.int32) + NP, pad_src])
    dst = jnp.concatenate([edge_index1[1].astype(jnp.int32), pad_dst,
                           edge_index2[1].astype(jnp.int32), pad_dst])
    zrows = jnp.zeros((NP - N, D), jnp.float32)
    x_both = jnp.concatenate([x1, zrows, x2, zrows])
    b0r = b0.reshape(1, D)
    b1r = b1.reshape(1, D)

    degf = _agg_call(jnp.ones((2 * NP, D), jnp.float32), src, dst)
    y0 = _mm_scale(x_both, W0, degf)
    agg0 = _agg_call(y0, src, dst)
    y1 = _mid(agg0, degf, b0r, W1)
    agg1 = _agg_call(y1, src, dst)
    z = _final(agg1, degf, b1r)
    return z[:N], z[NP:NP + N]
